# Initial kernel scaffold; baseline (speedup 1.0000x reference)
import jax
import jax.numpy as jnp
from jax.experimental import pallas as pl


def kernel(hidden_states, W_router, W_gate, W_up, W_down):
    raise NotImplementedError("write your pallas kernel here")


# speedup vs baseline: 1.8769x; 1.8769x over previous
import functools

import jax
import jax.numpy as jnp
from jax.experimental import pallas as pl
from jax.experimental.pallas import tpu as pltpu

NUM_EXPERTS = 8
TOP_K = 2
D_MODEL = 1024
D_FF = 2048
F_TILE = 1024
NF = D_FF // F_TILE


def _moe_dense_kernel(x_ref, wr_ref, wg_ref, wu_ref, wd_ref, out_ref, w_scr):
    e = pl.program_id(0)
    f = pl.program_id(1)

    @pl.when((e == 0) & (f == 0))
    def _routing():
        x = x_ref[...]
        logits = jnp.dot(x, wr_ref[...], preferred_element_type=jnp.float32)
        eidx = jax.lax.broadcasted_iota(jnp.int32, logits.shape, 1)
        m1 = jnp.max(logits, axis=1, keepdims=True)
        i1 = jnp.min(jnp.where(logits == m1, eidx, NUM_EXPERTS), axis=1, keepdims=True)
        neg = jnp.finfo(jnp.float32).min
        masked = jnp.where(eidx == i1, neg, logits)
        m2 = jnp.max(masked, axis=1, keepdims=True)
        i2 = jnp.min(jnp.where(masked == m2, eidx, NUM_EXPERTS), axis=1, keepdims=True)
        p2 = 1.0 / (1.0 + jnp.exp(m1 - m2))
        p1 = 1.0 - p2
        w_scr[...] = p1 * (eidx == i1) + p2 * (eidx == i2)

    x = x_ref[...]
    g = jnp.dot(x, wg_ref[0], preferred_element_type=jnp.float32)
    u = jnp.dot(x, wu_ref[0], preferred_element_type=jnp.float32)
    h = (g * jax.lax.logistic(g)) * u
    contrib = jnp.dot(h, wd_ref[0], preferred_element_type=jnp.float32)
    onehot = (jax.lax.broadcasted_iota(jnp.int32, (NUM_EXPERTS, 1), 0) == e
              ).astype(jnp.float32)
    w_col = jnp.dot(w_scr[...], onehot, preferred_element_type=jnp.float32)
    val = w_col * contrib

    @pl.when((e == 0) & (f == 0))
    def _init():
        out_ref[...] = val

    @pl.when((e > 0) | (f > 0))
    def _acc():
        out_ref[...] += val


def kernel(hidden_states, W_router, W_gate, W_up, W_down):
    batch, seq_len, hidden = hidden_states.shape
    flat = hidden_states.reshape(-1, hidden)
    T = flat.shape[0]

    out = pl.pallas_call(
        _moe_dense_kernel,
        grid=(NUM_EXPERTS, NF),
        in_specs=[
            pl.BlockSpec((T, D_MODEL), lambda e, f: (0, 0)),
            pl.BlockSpec((D_MODEL, NUM_EXPERTS), lambda e, f: (0, 0)),
            pl.BlockSpec((1, D_MODEL, F_TILE), lambda e, f: (e, 0, f)),
            pl.BlockSpec((1, D_MODEL, F_TILE), lambda e, f: (e, 0, f)),
            pl.BlockSpec((1, F_TILE, D_MODEL), lambda e, f: (e, f, 0)),
        ],
        out_specs=pl.BlockSpec((T, D_MODEL), lambda e, f: (0, 0)),
        out_shape=jax.ShapeDtypeStruct((T, D_MODEL), jnp.float32),
        scratch_shapes=[pltpu.VMEM((T, NUM_EXPERTS), jnp.float32)],
    )(flat, W_router, W_gate, W_up, W_down)
    return out.reshape(batch, seq_len, hidden)
